# baseline (speedup 1.0000x reference)
import functools

import jax
import jax.numpy as jnp
from jax import lax
from jax.experimental import pallas as pl
from jax.experimental.pallas import tpu as pltpu
from jax.experimental.pallas import tpu_sc as plsc

NB = 16
NF = 80
NT = 8192
NS = 512
MAXD = 15
ROWS_PER_W = 40
CH = 5
NCHUNK = ROWS_PER_W // CH

_mesh = plsc.VectorSubcoreMesh(core_axis_name="c", subcore_axis_name="s")

_row_buf_types = [pltpu.VMEM((NT + 16,), jnp.float32) for _ in range(2 * CH)]


@functools.partial(
    pl.kernel,
    mesh=_mesh,
    compiler_params=pltpu.CompilerParams(needs_layout_passes=False),
    out_type=jax.ShapeDtypeStruct((NB, NF, NS), jnp.float32),
    scratch_types=[
        pltpu.VMEM((NS,), jnp.int32),
        pltpu.VMEM((NS,), jnp.int32),
        pltpu.VMEM((ROWS_PER_W, NS), jnp.float32),
        *_row_buf_types,
        pltpu.SemaphoreType.DMA,
        pltpu.SemaphoreType.DMA,
    ],
)
def _seg_avg(x_hbm, dur_hbm, out_hbm, dur_v, starts_v, out_v, *bufs_and_sems):
    bufs = bufs_and_sems[: 2 * CH]
    sem_a, sem_b = bufs_and_sems[2 * CH :]
    bank_a, bank_b = bufs[:CH], bufs[CH:]

    cid = lax.axis_index("c")
    sid = lax.axis_index("s")
    wid = sid * 2 + cid
    batch = wid // 2
    f0 = (wid % 2) * ROWS_PER_W

    pltpu.sync_copy(dur_hbm.at[batch], dur_v)

    iota = jnp.arange(16, dtype=jnp.int32)
    lane15 = jnp.full((16,), 15, jnp.int32)

    def bounds_body(g, carry_v):
        base = pl.multiple_of(g * 16, 16)
        d = dur_v[pl.ds(base, 16)]
        ends = d
        for k in (1, 2, 4, 8):
            sh = ends.at[jnp.maximum(iota - k, 0)].get(mode="promise_in_bounds")
            ends = ends + jnp.where(iota >= k, sh, 0)
        ends = ends + carry_v
        starts_v[pl.ds(base, 16)] = ends - d
        return ends.at[lane15].get(mode="promise_in_bounds")

    lax.fori_loop(0, NS // 16, bounds_body, jnp.zeros((16,), jnp.int32))

    zeros = jnp.zeros((16,), jnp.float32)
    for b in bufs:
        b[pl.ds(NT, 16)] = zeros

    def compute_row(row_v, r):
        def grp_body(g, _2):
            base = pl.multiple_of(g * 32, 16)
            s0 = starts_v[pl.ds(base, 16)]
            l0 = dur_v[pl.ds(base, 16)]
            s1 = starts_v[pl.ds(base + 16, 16)]
            l1 = dur_v[pl.ds(base + 16, 16)]
            acc0 = cnt0 = acc1 = cnt1 = zeros
            for d in range(MAXD):
                i0 = jnp.where(l0 > d, s0 + d, NT)
                i1 = jnp.where(l1 > d, s1 + d, NT)
                v0 = plsc.load_gather(row_v, [i0])
                v1 = plsc.load_gather(row_v, [i1])
                acc0 = acc0 + v0
                acc1 = acc1 + v1
                cnt0 = cnt0 + jnp.where(v0 == 0.0, 0.0, 1.0)
                cnt1 = cnt1 + jnp.where(v1 == 0.0, 0.0, 1.0)
            out_v[r, pl.ds(base, 16)] = acc0 / jnp.maximum(cnt0, 1.0)
            out_v[r, pl.ds(base + 16, 16)] = acc1 / jnp.maximum(cnt1, 1.0)
            return 0

        lax.fori_loop(0, NS // 32, grp_body, 0)

    def fire(bank, c, sem):
        for j, b in enumerate(bank):
            pltpu.async_copy(
                x_hbm.at[batch, f0 + c * CH + j], b.at[pl.ds(0, NT)], sem
            )

    def drain(bank, c, sem):
        for j, b in enumerate(bank):
            pltpu.make_async_copy(
                x_hbm.at[batch, f0 + c * CH + j], b.at[pl.ds(0, NT)], sem
            ).wait()

    def compute_bank(bank, c):
        for j, b in enumerate(bank):
            compute_row(b, c * CH + j)

    fire(bank_a, 0, sem_a)

    def pair_body(p, _):
        ca = 2 * p
        fire(bank_b, ca + 1, sem_b)
        drain(bank_a, ca, sem_a)
        compute_bank(bank_a, ca)

        @pl.when(p < NCHUNK // 2 - 1)
        def _prefetch():
            fire(bank_a, ca + 2, sem_a)

        drain(bank_b, ca + 1, sem_b)
        compute_bank(bank_b, ca + 1)
        return 0

    lax.fori_loop(0, NCHUNK // 2, pair_body, 0)
    pltpu.sync_copy(out_v, out_hbm.at[batch, pl.ds(f0, ROWS_PER_W)])


def kernel(x, durations):
    return _seg_avg(x, durations.astype(jnp.int32))
